# baseline (speedup 1.0000x reference)
import functools
import jax
import jax.numpy as jnp
from jax import lax
from jax.experimental import pallas as pl
from jax.experimental.pallas import tpu as pltpu, tpu_sc as plsc

N = 2048
F_IN = 16
F_OUT = 32
B = 32
SEG = 64
KCH = 16

_mesh = plsc.VectorSubcoreMesh(core_axis_name="c", subcore_axis_name="s",
                               num_cores=2, num_subcores=16)


@functools.partial(
    pl.kernel,
    out_type=jax.ShapeDtypeStruct((N * 8,), jnp.float32),
    mesh=_mesh,
    compiler_params=pltpu.CompilerParams(needs_layout_passes=False),
    scratch_types=[
        pltpu.VMEM((N,), jnp.float32),
        pltpu.VMEM((N,), jnp.float32),
        pltpu.VMEM((N,), jnp.int32),
        pltpu.VMEM((N,), jnp.float32),
        pltpu.VMEM((N,), jnp.int32),
        pltpu.VMEM((N,), jnp.float32),
        pltpu.VMEM((KCH * SEG,), jnp.int32),
        pltpu.VMEM((KCH * SEG,), jnp.float32),
        pltpu.VMEM((KCH * SEG,), jnp.float32),
        pltpu.VMEM((KCH * SEG,), jnp.float32),
        pltpu.VMEM((SEG,), jnp.float32),
        pltpu.VMEM((SEG,), jnp.float32),
        pltpu.VMEM((SEG,), jnp.float32),
        pltpu.VMEM((SEG,), jnp.float32),
        pltpu.VMEM((SEG * 8,), jnp.float32),
        pltpu.SemaphoreType.DMA,
        pltpu.SemaphoreType.DMA,
    ],
)
def _build_c(g0f, g1f, c0_hbm, c1_hbm, out_hbm,
             c0_v, c1_v, m0, v0, m1, v1, idx_buf, vrep_buf, buf0, buf1,
             d00, d01, d10, d11, cseg, sem0, sem1):
    wid = lax.axis_index("s") * 2 + lax.axis_index("c")
    seg = wid * SEG

    pltpu.sync_copy(c0_hbm, c0_v)
    pltpu.sync_copy(c1_hbm, c1_v)

    iota = lax.iota(jnp.int32, 16)
    zf = jnp.zeros((16,), jnp.float32)
    zi = jnp.zeros((16,), jnp.int32)

    for u in range(SEG // 16):
        sl = pl.ds(u * 16, 16)
        d00[sl] = zf
        d01[sl] = zf
        d10[sl] = zf
        d11[sl] = zf

    def scan_col(cv_ref, m_ref, v_ref):
        def body(i, cnt):
            v = cv_ref[pl.ds(i * 16, 16)]
            msk = v != 0.0
            pos = cnt + plsc.cumsum(msk.astype(jnp.int32)) - 1
            plsc.store_scatter(m_ref, [pos], iota + i * 16, mask=msk)
            plsc.store_scatter(v_ref, [pos], v, mask=msk)
            return cnt + plsc.all_reduce_population_count(msk)
        return lax.fori_loop(0, N // 16, body, zi)

    cnt0 = scan_col(c0_v, m0, v0)
    cnt1 = scan_col(c1_v, m1, v1)

    def do_col(m_ref, v_ref, cnt_vec, dA, dB):
        nnz = jnp.max(cnt_vec)
        nch = (nnz + KCH - 1) // KCH

        def chunk(cI, _):
            for u in range(KCH * SEG // 16):
                kv = zi + (cI * KCH + u // 4)
                vmask = kv < cnt_vec
                m_rep = jnp.where(vmask, plsc.load_gather(m_ref, [kv]), 0)
                v_rep = jnp.where(vmask, plsc.load_gather(v_ref, [kv]), 0.0)
                j = (u % 4) * 16 + iota
                idx_buf[pl.ds(u * 16, 16)] = (seg + j) * N + m_rep
                vrep_buf[pl.ds(u * 16, 16)] = v_rep
            h0 = pltpu.async_copy(g0f.at[idx_buf], buf0, sem0)
            h1 = pltpu.async_copy(g1f.at[idx_buf], buf1, sem1)
            h0.wait()
            h1.wait()
            for u in range(KCH * SEG // 16):
                sl = pl.ds((u % 4) * 16, 16)
                bsl = pl.ds(u * 16, 16)
                vr = vrep_buf[bsl]
                dA[sl] += buf0[bsl] * vr
                dB[sl] += buf1[bsl] * vr
            return 0

        lax.fori_loop(0, nch, chunk, 0)

    do_col(m0, v0, cnt0, d00, d10)
    do_col(m1, v1, cnt1, d01, d11)

    for u in range(SEG // 16):
        base = (iota + u * 16) * 8
        usl = pl.ds(u * 16, 16)
        plsc.store_scatter(cseg, [base + 0], c0_v[pl.ds(seg + u * 16, 16)])
        plsc.store_scatter(cseg, [base + 1], c1_v[pl.ds(seg + u * 16, 16)])
        plsc.store_scatter(cseg, [base + 2], d00[usl])
        plsc.store_scatter(cseg, [base + 3], d01[usl])
        plsc.store_scatter(cseg, [base + 4], d10[usl])
        plsc.store_scatter(cseg, [base + 5], d11[usl])
        plsc.store_scatter(cseg, [base + 6], zf)
        plsc.store_scatter(cseg, [base + 7], zf)
    pltpu.sync_copy(cseg, out_hbm.at[pl.ds(wid * SEG * 8, SEG * 8)])


def _tc_body(C_ref, xb_ref, x0_ref, W0_ref, W6_ref, b2_ref,
             R0w_ref, R0b_ref, R1w_ref, R1b_ref, out_ref, Y_ref):
    bidx = pl.program_id(0)
    s = lax.dot_general(C_ref[...], xb_ref[...], (((0,), (0,)), ((), ())),
                        preferred_element_type=jnp.float32)
    w6 = W6_ref[...]
    yb = jnp.dot(x0_ref[pl.ds(bidx, 1), :], W0_ref[...],
                 preferred_element_type=jnp.float32)
    for t in range(6):
        yb = yb + jnp.dot(s[t:t + 1, :], w6[t],
                          preferred_element_type=jnp.float32)
    Y_ref[pl.ds(bidx, 1), :] = yb

    @pl.when(bidx == B - 1)
    def _final():
        y = jax.nn.sigmoid(Y_ref[...] + b2_ref[...])
        h = jax.nn.sigmoid(jnp.dot(y, R0w_ref[...],
                                   preferred_element_type=jnp.float32) + R0b_ref[...])
        z = jnp.dot(h, R1w_ref[...],
                    preferred_element_type=jnp.float32) + R1b_ref[...]
        out_ref[...] = 1.0 + 4.0 * jax.nn.sigmoid(z)


def kernel(x, G0, G1, W, b, R0_w, R0_b, R1_w, R1_b):
    c0 = G0[:, 0]
    c1 = G1[:, 0]
    Cflat = _build_c(G0.reshape(-1), G1.reshape(-1), c0, c1)
    C = Cflat.reshape(N, 8)

    full = lambda s: pl.BlockSpec(s, lambda i: tuple(0 for _ in s))
    out = pl.pallas_call(
        _tc_body,
        grid=(B,),
        in_specs=[
            full((N, 8)),
            pl.BlockSpec((None, N, F_IN), lambda i: (i, 0, 0)),
            full((B, F_IN)),
            full((F_IN, F_OUT)),
            full((6, F_IN, F_OUT)),
            full((1, F_OUT)),
            full((F_OUT, 16)),
            full((1, 16)),
            full((16, 1)),
            full((1, 1)),
        ],
        out_specs=pl.BlockSpec((B, 1), lambda i: (0, 0)),
        out_shape=jax.ShapeDtypeStruct((B, 1), jnp.float32),
        scratch_shapes=[pltpu.VMEM((B, F_OUT), jnp.float32)],
    )(C, x, x[:, 0, :], W[0], W[1:7], b.reshape(1, F_OUT),
      R0_w, R0_b.reshape(1, 16), R1_w, R1_b.reshape(1, 1))
    return out


# speedup vs baseline: 1.4660x; 1.4660x over previous
import jax
import jax.numpy as jnp
from jax import lax
from jax.experimental import pallas as pl
from jax.experimental.pallas import tpu as pltpu

N = 2048
F_IN = 16
F_OUT = 32
B = 32
KB = 8
NKB = N // KB


def _body(g0_ref, g1_ref, cv_ref, xb_ref, x0_ref, W0_ref, W6_ref, b2_ref,
          R0w_ref, R0b_ref, R1w_ref, R1b_ref, out_ref, C_ref, Y_ref):
    i = pl.program_id(0)

    @pl.when(i == 0)
    def _init():
        C_ref[:, 0:2] = cv_ref[...]
        C_ref[:, 6:8] = jnp.zeros((N, 2), jnp.float32)

    @pl.when(i < KB)
    def _matvec():
        cvb = cv_ref[pl.ds(i * NKB, NKB), :]
        d0 = jnp.dot(g0_ref[...], cvb, preferred_element_type=jnp.float32)
        d1 = jnp.dot(g1_ref[...], cvb, preferred_element_type=jnp.float32)

        @pl.when(i == 0)
        def _set():
            C_ref[:, 2:4] = d0
            C_ref[:, 4:6] = d1

        @pl.when(i > 0)
        def _acc():
            C_ref[:, 2:4] += d0
            C_ref[:, 4:6] += d1

    @pl.when(i >= KB)
    def _contract():
        bidx = i - KB
        s = lax.dot_general(C_ref[...], xb_ref[...], (((0,), (0,)), ((), ())),
                            preferred_element_type=jnp.float32)
        w6 = W6_ref[...]
        yb = jnp.dot(x0_ref[pl.ds(bidx, 1), :], W0_ref[...],
                     preferred_element_type=jnp.float32)
        for t in range(6):
            yb = yb + jnp.dot(s[t:t + 1, :], w6[t],
                              preferred_element_type=jnp.float32)
        Y_ref[pl.ds(bidx, 1), :] = yb

    @pl.when(i == KB + B - 1)
    def _final():
        y = jax.nn.sigmoid(Y_ref[...] + b2_ref[...])
        h = jax.nn.sigmoid(jnp.dot(y, R0w_ref[...],
                                   preferred_element_type=jnp.float32) + R0b_ref[...])
        z = jnp.dot(h, R1w_ref[...],
                    preferred_element_type=jnp.float32) + R1b_ref[...]
        out_ref[...] = 1.0 + 4.0 * jax.nn.sigmoid(z)


def kernel(x, G0, G1, W, b, R0_w, R0_b, R1_w, R1_b):
    cvec = jnp.stack([G0[:, 0], G1[:, 0]], axis=1)

    full = lambda s: pl.BlockSpec(s, lambda i: tuple(0 for _ in s))
    out = pl.pallas_call(
        _body,
        grid=(KB + B,),
        in_specs=[
            pl.BlockSpec((N, NKB), lambda i: (0, jnp.minimum(i, KB - 1))),
            pl.BlockSpec((N, NKB), lambda i: (0, jnp.minimum(i, KB - 1))),
            full((N, 2)),
            pl.BlockSpec((None, N, F_IN),
                         lambda i: (jnp.maximum(i - KB, 0), 0, 0)),
            full((B, F_IN)),
            full((F_IN, F_OUT)),
            full((6, F_IN, F_OUT)),
            full((1, F_OUT)),
            full((F_OUT, 16)),
            full((1, 16)),
            full((16, 1)),
            full((1, 1)),
        ],
        out_specs=pl.BlockSpec((B, 1), lambda i: (0, 0)),
        out_shape=jax.ShapeDtypeStruct((B, 1), jnp.float32),
        scratch_shapes=[pltpu.VMEM((N, 8), jnp.float32),
                        pltpu.VMEM((B, F_OUT), jnp.float32)],
    )(G0, G1, cvec, x, x[:, 0, :], W[0], W[1:7], b.reshape(1, F_OUT),
      R0_w, R0_b.reshape(1, 16), R1_w, R1_b.reshape(1, 1))
    return out


# speedup vs baseline: 3.7848x; 2.5817x over previous
import jax
import jax.numpy as jnp
from jax.experimental import pallas as pl

N = 2048
F_IN = 16
F_OUT = 32
B = 32

_HI = jax.lax.Precision.HIGHEST


def _body(g0, g1, xtb, x0T, W0T, W6T, bcol, R0T, R0b, R1T, R1b, out):
    c0 = g0[:, 0:1]
    c1 = g1[:, 0:1]
    cv = jnp.concatenate([c0, c1], axis=1)
    d0 = jnp.dot(g0[...], cv, preferred_element_type=jnp.float32)
    d1 = jnp.dot(g1[...], cv, preferred_element_type=jnp.float32)
    zero2 = jnp.zeros((N, 2), jnp.float32)
    C = jnp.concatenate([cv, d0, d1, zero2], axis=1)
    a = jax.lax.dot_general(C, xtb[...], (((0,), (0,)), ((), ())),
                            preferred_element_type=jnp.float32)
    yT = jnp.dot(W0T[...], x0T[...],
                 preferred_element_type=jnp.float32)
    for f in range(F_IN):
        yT += jnp.dot(W6T[f], a[:, f * B:(f + 1) * B],  preferred_element_type=jnp.float32)
    yT = jax.nn.sigmoid(yT + bcol[...])
    h = jax.nn.sigmoid(jnp.dot(R0T[...], yT,           preferred_element_type=jnp.float32) + R0b[...])
    z = jnp.dot(R1T[...], h,
                preferred_element_type=jnp.float32) + R1b[...]
    out[...] = 1.0 + 4.0 * jax.nn.sigmoid(z)


def kernel(x, G0, G1, W, b, R0_w, R0_b, R1_w, R1_b):
    xt = jnp.transpose(x, (1, 2, 0)).reshape(N, F_IN * B)
    x0T = x[:, 0, :].T
    W0T = W[0].T
    W6T = jnp.concatenate(
        [jnp.transpose(W[1:7], (1, 2, 0)),
         jnp.zeros((F_IN, F_OUT, 2), jnp.float32)], axis=2)
    bcol = b.reshape(F_OUT, 1)
    R0T = R0_w.T
    R0b = R0_b.reshape(16, 1)
    R1T = R1_w.T
    R1b = R1_b.reshape(1, 1)

    outT = pl.pallas_call(
        _body,
        out_shape=jax.ShapeDtypeStruct((1, B), jnp.float32),
    )(G0, G1, xt, x0T, W0T, W6T, bcol, R0T, R0b, R1T, R1b)
    return outT.reshape(B, 1)
